# baseline (speedup 1.0000x reference)
import jax
import jax.numpy as jnp
from jax import lax
from jax.experimental import pallas as pl
from jax.experimental.pallas import tpu as pltpu
from jax.experimental.pallas import tpu_sc as plsc

_NUM_CORES = 2
_NUM_SUBCORES = 16
_NUM_WORKERS = _NUM_CORES * _NUM_SUBCORES
_LANES = 16

_BATCH = 4096
_SEQ = 200
_EMBED = 64
_PT = _SEQ // 8
_BT = _BATCH // 128
_TP = 133
_NG = 7
_NT = 3
_LAG = 5


def _sc_body(x_hbm, pe_hbm, table_hbm, out_hbm, idx_v, pe_v, g_v, t_v,
             sem_g, sem_s):
  w = lax.axis_index("s") * _NUM_CORES + lax.axis_index("c")

  pltpu.sync_copy(x_hbm.at[:, w], idx_v)
  pltpu.sync_copy(pe_hbm, pe_v)

  iota = lax.iota(jnp.int32, _LANES)
  idx_dr = [(16 * j + iota) // 8 for j in range(_EMBED // _LANES)]
  idx_sl = [lax.rem(16 * j + iota, 8) for j in range(_EMBED // _LANES)]

  def gather(p):
    pltpu.async_copy(
        table_hbm.at[idx_v.at[p // 8, lax.rem(p, 8)]],
        g_v.at[lax.rem(p, _NG)], sem_g)

  def drain_gather(p):
    pltpu.make_async_copy(
        table_hbm.at[idx_v.at[p // 8, lax.rem(p, 8)]],
        g_v.at[lax.rem(p, _NG)], sem_g).wait()

  def store(p):
    pltpu.async_copy(t_v.at[lax.rem(p, _NT), :, :, pl.ds(0, 128)],
                     out_hbm.at[p, :, w], sem_s)

  def drain_store():
    pltpu.make_async_copy(
        t_v.at[0, :, :, pl.ds(0, 128)], out_hbm.at[0, :, 0], sem_s).wait()

  for p in range(_LAG):
    gather(p)

  def body(p, _):
    @pl.when(p + _LAG < _SEQ)
    def _():
      gather(p + _LAG)

    drain_gather(p)

    @pl.when(p >= _NT)
    def _():
      drain_store()

    s = lax.rem(p, _NG)
    ts = lax.rem(p, _NT)
    t3 = t_v.at[ts]
    pvec = [pe_v[p, pl.ds(_LANES * j, _LANES)]
            for j in range(_EMBED // _LANES)]

    def token(i):
      lane_i = jnp.full((_LANES,), i, jnp.int32)
      for j in range(_EMBED // _LANES):
        vals = g_v[s, i, pl.ds(_LANES * j, _LANES)] + pvec[j]
        plsc.store_scatter(t3, [idx_dr[j], idx_sl[j], lane_i], vals)

    plsc.parallel_loop(0, 128, 1, unroll=8)(token)
    store(p)
    return 0

  lax.fori_loop(0, _SEQ, body, 0)

  for _ in range(_NT):
    drain_store()


@jax.jit
def _pe_lookup(x4, table2, pe):
  mesh = plsc.VectorSubcoreMesh(core_axis_name="c", subcore_axis_name="s")
  return pl.kernel(
      _sc_body,
      out_type=jax.ShapeDtypeStruct((_SEQ, 8, _BT, 8, 128), jnp.float32),
      mesh=mesh,
      scratch_types=[
          pltpu.VMEM((_PT, 8, 128), jnp.int32),
          pltpu.VMEM((_SEQ, _EMBED), jnp.float32),
          pltpu.VMEM((_NG, 128, _EMBED), jnp.float32),
          pltpu.VMEM((_NT, 8, 8, _TP), jnp.float32),
          pltpu.SemaphoreType.DMA,
          pltpu.SemaphoreType.DMA,
      ],
      compiler_params=pltpu.CompilerParams(
          use_tc_tiling_on_sc=False, needs_layout_passes=False),
  )(x4, pe, table2)


def kernel(x, table, pe):
  x4 = ((x.astype(jnp.int32) * 2).T
        .reshape(_PT, 8, _BT, 128).transpose(0, 2, 1, 3))
  table2 = jnp.pad(table, ((0, 0), (0, 64))).reshape(2 * 1000000, _EMBED)
  out5 = _pe_lookup(x4, table2, pe)
  return out5.transpose(2, 4, 0, 1, 3).reshape(_BATCH, _SEQ, _EMBED)


# speedup vs baseline: 1.0030x; 1.0030x over previous
import jax
import jax.numpy as jnp
from jax import lax
from jax.experimental import pallas as pl
from jax.experimental.pallas import tpu as pltpu
from jax.experimental.pallas import tpu_sc as plsc

_NUM_CORES = 2
_NUM_SUBCORES = 16
_NUM_WORKERS = _NUM_CORES * _NUM_SUBCORES
_LANES = 16

_BATCH = 4096
_SEQ = 200
_EMBED = 64
_PT = _SEQ // 8
_BT = _BATCH // 128
_TP = 133
_NG = 7
_NT = 3
_LAG = 5


def _sc_body(x_hbm, pe_hbm, table_hbm, out_hbm, idx_v, pe_v, g_v, t_v,
             sem_g, sem_s):
  w = lax.axis_index("s") * _NUM_CORES + lax.axis_index("c")

  pltpu.sync_copy(x_hbm.at[:, w], idx_v)
  pltpu.sync_copy(pe_hbm, pe_v)

  iota = lax.iota(jnp.int32, _LANES)
  idx_dr = [(16 * j + iota) // 8 for j in range(_EMBED // _LANES)]
  idx_sl = [lax.rem(16 * j + iota, 8) for j in range(_EMBED // _LANES)]

  def gather(p):
    pltpu.async_copy(
        table_hbm.at[idx_v.at[p // 8, lax.rem(p, 8)]],
        g_v.at[lax.rem(p, _NG)], sem_g)

  def drain_gather(p):
    pltpu.make_async_copy(
        table_hbm.at[idx_v.at[p // 8, lax.rem(p, 8)]],
        g_v.at[lax.rem(p, _NG)], sem_g).wait()

  def store(p):
    pltpu.async_copy(t_v.at[lax.rem(p, _NT), :, :, pl.ds(0, 128)],
                     out_hbm.at[p, :, w], sem_s)

  def drain_store():
    pltpu.make_async_copy(
        t_v.at[0, :, :, pl.ds(0, 128)], out_hbm.at[0, :, 0], sem_s).wait()

  for p in range(_LAG):
    gather(p)

  def body(p, _):
    @pl.when(p + _LAG < _SEQ)
    def _():
      gather(p + _LAG)

    drain_gather(p)

    @pl.when(p >= _NT)
    def _():
      drain_store()

    s = lax.rem(p, _NG)
    ts = lax.rem(p, _NT)
    t3 = t_v.at[ts]
    pvec = [pe_v[p, pl.ds(_LANES * j, _LANES)]
            for j in range(_EMBED // _LANES)]

    def token(i):
      lane_i = jnp.full((_LANES,), i, jnp.int32)
      for j in range(_EMBED // _LANES):
        vals = g_v[s, i, pl.ds(_LANES * j, _LANES)] + pvec[j]
        plsc.store_scatter(t3, [idx_dr[j], idx_sl[j], lane_i], vals)

    plsc.parallel_loop(0, 128, 1, unroll=4)(token)
    store(p)
    return 0

  lax.fori_loop(0, _SEQ, body, 0)

  for _ in range(_NT):
    drain_store()


@jax.jit
def _pe_lookup(x4, table2, pe):
  mesh = plsc.VectorSubcoreMesh(core_axis_name="c", subcore_axis_name="s")
  return pl.kernel(
      _sc_body,
      out_type=jax.ShapeDtypeStruct((_SEQ, 8, _BT, 8, 128), jnp.float32),
      mesh=mesh,
      scratch_types=[
          pltpu.VMEM((_PT, 8, 128), jnp.int32),
          pltpu.VMEM((_SEQ, _EMBED), jnp.float32),
          pltpu.VMEM((_NG, 128, _EMBED), jnp.float32),
          pltpu.VMEM((_NT, 8, 8, _TP), jnp.float32),
          pltpu.SemaphoreType.DMA,
          pltpu.SemaphoreType.DMA,
      ],
      compiler_params=pltpu.CompilerParams(
          use_tc_tiling_on_sc=False, needs_layout_passes=False),
  )(x4, pe, table2)


def kernel(x, table, pe):
  x4 = ((x.astype(jnp.int32) * 2).T
        .reshape(_PT, 8, _BT, 128).transpose(0, 2, 1, 3))
  table2 = jnp.pad(table, ((0, 0), (0, 64))).reshape(2 * 1000000, _EMBED)
  out5 = _pe_lookup(x4, table2, pe)
  return out5.transpose(2, 4, 0, 1, 3).reshape(_BATCH, _SEQ, _EMBED)
